# Initial kernel scaffold; baseline (speedup 1.0000x reference)
import jax
import jax.numpy as jnp
from jax.experimental import pallas as pl


def kernel(obs_encoding, lane_encoding, same_obs_mask):
    raise NotImplementedError("write your pallas kernel here")


# speedup vs baseline: 6.2053x; 6.2053x over previous
import jax
import jax.numpy as jnp
from jax import lax
from jax.experimental import pallas as pl
from jax.experimental.pallas import tpu as pltpu
from jax.experimental.pallas import tpu_sc as plsc

N = 10000
M = 320000
D = 128
OD = 2 * D
NW = 32
SEGW = 320
LASTW = N - (NW - 1) * SEGW
CH = 128
KV = D // 16
NB = 48


def _seg_agg(seg_hbm, lane_hbm, bounds_hbm, out_hbm, bounds_v, sbuf, lbuf, stage):
    c = lax.axis_index("c")
    s = lax.axis_index("s")
    w = s * 2 + c

    pltpu.sync_copy(bounds_hbm, bounds_v)
    bv = bounds_v[pl.ds(w, 16)]
    rs = bv[0]
    re = bv[1]
    lo = w * SEGW

    zeros = jnp.zeros((16,), jnp.float32)

    def zero_blk(i, carry):
        for k in range(8):
            stage[pl.ds(i * 128 + k * 16, 16)] = zeros
        return carry

    lax.fori_loop(0, SEGW * OD // 128, zero_blk, 0)

    neg_inf = jnp.full((16,), -jnp.inf, jnp.float32)

    def flush(cur, cnt, mx, sm):
        r = (cur - lo) * OD
        inv = jnp.ones((16,), jnp.float32) / jnp.broadcast_to(cnt, (16,))
        for k in range(KV):
            stage[pl.ds(r + k * 16, 16)] = mx[k]
        for k in range(KV):
            stage[pl.ds(r + D + k * 16, 16)] = sm[k] * inv

    def step(sid, row, rc):
        cur, cnt, mx, sm = rc
        is_new = sid != cur

        @pl.when(jnp.logical_and(is_new, cnt > 0.0))
        def _():
            flush(cur, cnt, mx, sm)

        nmx = tuple(
            jnp.where(is_new, row[k], jnp.maximum(mx[k], row[k]))
            for k in range(KV))
        nsm = tuple(
            jnp.where(is_new, row[k], sm[k] + row[k])
            for k in range(KV))
        ncnt = jnp.where(is_new, 1.0, cnt + 1.0)
        return sid, ncnt, nmx, nsm

    skip = rs - (rs // 8) * 8
    rs8 = rs - skip
    nchunks = (re - rs8 + CH - 1) // CH

    def chunk_body(g, carry):
        cstart = rs8 + g * CH
        nrows = jnp.minimum(CH, re - cstart)
        i0 = jnp.where(g == 0, skip, 0)
        base = jnp.minimum(cstart, M - CH)
        woff = cstart - base
        sbase = jnp.minimum(cstart, M - CH - 16)
        soff = cstart - sbase
        pltpu.sync_copy(lane_hbm.at[pl.ds(base * D, CH * D)], lbuf)
        pltpu.sync_copy(seg_hbm.at[pl.ds(sbase, CH + 16)], sbuf.at[pl.ds(0, CH + 16)])

        nvalid = nrows - i0
        ngroups = nvalid // 16

        def group_body(gi, rc):
            rbase = i0 + gi * 16
            segv = sbuf[pl.ds(soff + rbase, 16)]
            roff = (woff + rbase) * D
            for j in range(16):
                row = tuple(
                    lbuf[pl.ds(roff + j * D + k * 16, 16)] for k in range(KV))
                rc = step(segv[j], row, rc)
            return rc

        rc = lax.fori_loop(0, ngroups, group_body, carry)

        def tail_body(t, rc):
            ri = i0 + ngroups * 16 + t
            sid = sbuf[pl.ds(soff + ri, 16)][0]
            roff = (woff + ri) * D
            row = tuple(lbuf[pl.ds(roff + k * 16, 16)] for k in range(KV))
            return step(sid, row, rc)

        return lax.fori_loop(0, nvalid - ngroups * 16, tail_body, rc)

    cur, cnt, mx, sm = lax.fori_loop(
        0, nchunks, chunk_body,
        (jnp.int32(-1), jnp.float32(0.0), (neg_inf,) * KV, (zeros,) * KV))

    @pl.when(cnt > 0.0)
    def _():
        flush(cur, cnt, mx, sm)

    @pl.when(w < NW - 1)
    def _():
        pltpu.sync_copy(stage, out_hbm.at[pl.ds(lo * OD, SEGW * OD)])

    @pl.when(w == NW - 1)
    def _():
        pltpu.sync_copy(stage.at[pl.ds(0, LASTW * OD)],
                        out_hbm.at[pl.ds(lo * OD, LASTW * OD)])


def kernel(obs_encoding, lane_encoding, same_obs_mask):
    del obs_encoding
    seg = same_obs_mask.reshape(M)
    qs = jnp.minimum(jnp.arange(NW + 1, dtype=jnp.int32) * SEGW, N)
    bounds = jnp.searchsorted(seg, qs, side="left").astype(jnp.int32)
    bounds = jnp.concatenate([bounds, jnp.zeros((NB - (NW + 1),), jnp.int32)])
    mesh = plsc.VectorSubcoreMesh(core_axis_name="c", subcore_axis_name="s")
    run = pl.kernel(
        _seg_agg,
        out_type=jax.ShapeDtypeStruct((N * OD,), jnp.float32),
        mesh=mesh,
        scratch_types=[
            pltpu.VMEM((NB,), jnp.int32),
            pltpu.VMEM((CH + 32,), jnp.int32),
            pltpu.VMEM((CH * D,), jnp.float32),
            pltpu.VMEM((SEGW * OD,), jnp.float32),
        ],
    )
    out = run(seg, lane_encoding.reshape(M * D), bounds)
    return out.reshape(N, OD)
